# baseline (speedup 1.0000x reference)
import functools

import jax
import jax.numpy as jnp
from jax import lax
from jax.experimental import pallas as pl
from jax.experimental.pallas import tpu as pltpu
from jax.experimental.pallas import tpu_sc as plsc

NC = 2
NS = 16
NW = NC * NS
LANES = 16


def _compute_y(x, weight, bias, bn):
    n, d_in = x.shape
    t, d_out, _ = weight.shape

    def body(x_ref, w_ref, b_ref, y_ref):
        for ti in range(t):
            y = lax.dot_general(
                x_ref[...], w_ref[ti],
                dimension_numbers=(((1,), (1,)), ((), ())),
                preferred_element_type=jnp.float32,
            )
            y_ref[ti] = y + b_ref[ti]

    return pl.pallas_call(
        body,
        grid=(n // bn,),
        in_specs=[
            pl.BlockSpec((bn, d_in), lambda i: (i, 0)),
            pl.BlockSpec((t, d_out, d_in), lambda i: (0, 0, 0)),
            pl.BlockSpec((t, 1, d_out), lambda i: (0, 0, 0)),
        ],
        out_specs=pl.BlockSpec((t, bn, d_out), lambda i: (0, i, 0)),
        out_shape=jax.ShapeDtypeStruct((t, n, d_out), jnp.float32),
    )(x, weight, bias.reshape(t, 1, d_out))


def _pidx_call(edge_index, edge_type, n_nodes):
    e = edge_type.shape[0]
    rows, cols = e // 1280, 1280
    ei3 = edge_index.reshape(2, rows, cols)
    et2 = edge_type.reshape(rows, cols)

    def body(src_ref, tgt_ref, et_ref, o_ref):
        o_ref[...] = ((et_ref[...] * n_nodes + src_ref[0]) * 16384
                      + tgt_ref[0])

    out = pl.pallas_call(
        body,
        grid=(1,),
        in_specs=[
            pl.BlockSpec((1, rows, cols), lambda i: (0, 0, 0)),
            pl.BlockSpec((1, rows, cols), lambda i: (1, 0, 0)),
            pl.BlockSpec((rows, cols), lambda i: (0, 0)),
        ],
        out_specs=pl.BlockSpec((rows, cols), lambda i: (0, 0)),
        out_shape=jax.ShapeDtypeStruct((rows, cols), jnp.int32),
    )(ei3, ei3, et2)
    return out.reshape(e)


def _sc_scatter(y2, pidx, n_nodes, d, ch):
    e = pidx.shape[0]
    half = n_nodes // NC
    slab = e // NS
    nvr = slab // LANES
    gmax = y2.shape[0] - 1
    full = (half // NS) & ~7
    rem = half - NS * full
    assert full % 8 == 0 and rem % 8 == 0 and (full % 40) % 8 == 0 and rem <= 40
    assert e % NS == 0 and slab % LANES == 0 and ch % LANES == 0
    mesh = plsc.VectorSubcoreMesh(
        core_axis_name="c", subcore_axis_name="s", num_cores=NC, num_subcores=NS)

    @functools.partial(
        pl.kernel,
        mesh=mesh,
        out_type=jax.ShapeDtypeStruct((n_nodes, d), jnp.float32),
        compiler_params=pltpu.CompilerParams(needs_layout_passes=False),
        scratch_types=[
            pltpu.VMEM((slab + 4 * ch,), jnp.int32),
            pltpu.VMEM((ch,), jnp.int32),
            pltpu.VMEM((ch,), jnp.int32),
            pltpu.VMEM((ch,), jnp.int32),
            pltpu.VMEM((ch,), jnp.int32),
            pltpu.VMEM((ch, d), jnp.float32),
            pltpu.VMEM((ch, d), jnp.float32),
            pltpu.VMEM((40, d), jnp.float32),
            pltpu.VMEM_SHARED((half + 8, d), jnp.float32),
            pltpu.SemaphoreType.DMA,
            pltpu.SemaphoreType.DMA,
            pltpu.SemaphoreType.DMA,
            pltpu.SemaphoreType.DMA,
            pltpu.SemaphoreType.DMA,
            pltpu.SemaphoreType.DMA,
        ],
    )
    def k(y_h, p_h, out_h, cpvm, g0, t0, g1, t1, rows0, rows1, zbuf, acc,
          sem0, sem1, ssc0, ssc1, semp, semz):
        c = lax.axis_index("c")
        s = lax.axis_index("s")
        pdma = pltpu.async_copy(p_h.at[pl.ds(s * slab, slab)],
                                cpvm.at[pl.ds(0, slab)], semp)

        zero = jnp.zeros((LANES,), jnp.float32)
        def zrow_body(i, carry):
            for v in range(d // LANES):
                zbuf[i, pl.ds(v * LANES, LANES)] = zero
            return carry
        lax.fori_loop(0, 40, zrow_body, 0)

        a0 = s * full
        nz, zt = full // 40, full % 40
        for kk in range(nz):
            pltpu.async_copy(zbuf, acc.at[pl.ds(a0 + kk * 40, 40)], semz)
        if zt:
            pltpu.async_copy(zbuf.at[pl.ds(0, zt)],
                             acc.at[pl.ds(a0 + nz * 40, zt)], semz)
        @pl.when(s == NS - 1)
        def _zero_tail():
            pltpu.async_copy(zbuf.at[pl.ds(0, rem)],
                             acc.at[pl.ds(NS * full, rem)], semz)

        pdma.wait()
        lo = c * half
        hs = lo + half
        trash = slab + 4 * ch - 1
        def cbody(it, off):
            pv = cpvm[pl.ds(it * LANES, LANES)]
            tv = pv & 16383
            m = (tv >= lo) & (tv < hs)
            mi = m.astype(jnp.int32)
            pos = jnp.where(m, off + plsc.cumsum(mi) - 1, trash)
            plsc.store_scatter(cpvm, [jnp.clip(pos, 0, trash)], pv)
            return off + jnp.sum(mi)
        cnt = jnp.clip(lax.fori_loop(0, nvr, cbody, 0), 0, slab)

        q = jnp.clip(((cnt + ch - 1) // ch) | 1, 3, slab // ch + 1)
        padv = jnp.full((LANES,), hs * 16384 + hs, jnp.int32)
        lio = lax.iota(jnp.int32, LANES)
        for it in range(3 * ch // LANES + 1):
            plsc.store_scatter(cpvm, [jnp.clip(cnt + it * LANES + lio,
                                               0, trash - 1)], padv)

        def decode(j, gt, tt):
            for v in range(ch // LANES):
                pv = cpvm[pl.ds(j * ch + v * LANES, LANES)]
                gt[pl.ds(v * LANES, LANES)] = jnp.clip(pv >> 14, 0, gmax)
                tt[pl.ds(v * LANES, LANES)] = (
                    jnp.clip(pv & 16383, lo, hs) - lo)

        decode(0, g0, t0)
        pltpu.async_copy(y_h.at[g0], rows0, sem0)
        decode(1, g1, t1)
        pltpu.async_copy(y_h.at[g1], rows1, sem1)

        for kk in range(nz):
            pltpu.make_async_copy(zbuf, acc.at[pl.ds(a0 + kk * 40, 40)],
                                  semz).wait()
        if zt:
            pltpu.make_async_copy(zbuf.at[pl.ds(0, zt)],
                                  acc.at[pl.ds(a0 + nz * 40, zt)], semz).wait()
        @pl.when(s == NS - 1)
        def _zero_tail_wait():
            pltpu.make_async_copy(zbuf.at[pl.ds(0, rem)],
                                  acc.at[pl.ds(NS * full, rem)], semz).wait()
        plsc.subcore_barrier()

        pltpu.make_async_copy(y_h.at[g0], rows0, sem0).wait()
        pltpu.async_copy(rows0, acc.at[t0], ssc0, add=True)

        def pair(i, carry):
            j1 = 2 * i + 1
            @pl.when(j1 < q)
            def _unit_odd():
                pltpu.make_async_copy(rows0, acc.at[t0], ssc0).wait()
                @pl.when(j1 + 1 < q)
                def _():
                    decode(j1 + 1, g0, t0)
                    pltpu.async_copy(y_h.at[g0], rows0, sem0)
                pltpu.make_async_copy(y_h.at[g1], rows1, sem1).wait()
                pltpu.async_copy(rows1, acc.at[t1], ssc1, add=True)
            @pl.when(j1 + 1 < q)
            def _unit_even():
                pltpu.make_async_copy(rows1, acc.at[t1], ssc1).wait()
                @pl.when(j1 + 2 < q)
                def _():
                    decode(j1 + 2, g1, t1)
                    pltpu.async_copy(y_h.at[g1], rows1, sem1)
                pltpu.make_async_copy(y_h.at[g0], rows0, sem0).wait()
                pltpu.async_copy(rows0, acc.at[t0], ssc0, add=True)
            return carry
        lax.fori_loop(0, (slab // ch + 1 - 1) // 2, pair, 0)
        pltpu.make_async_copy(rows0, acc.at[t0], ssc0).wait()

        plsc.subcore_barrier()
        pltpu.sync_copy(acc.at[pl.ds(a0, full)],
                        out_h.at[pl.ds(c * half + a0, full)])
        @pl.when(s == NS - 1)
        def _write_tail():
            pltpu.sync_copy(acc.at[pl.ds(NS * full, rem)],
                            out_h.at[pl.ds(c * half + NS * full, rem)])

    return k(y2, pidx)


def kernel(x, edge_index, edge_type, weight, bias):
    n, d_in = x.shape
    t, d_out, _ = weight.shape
    y = _compute_y(x, weight, bias, bn=1000)
    y2 = y.reshape(t * n, d_out)
    pidx = _pidx_call(edge_index, edge_type, n_nodes=n)
    return _sc_scatter(y2, pidx, n_nodes=n, d=d_out, ch=128)


# speedup vs baseline: 1.5817x; 1.5817x over previous
import functools

import jax
import jax.numpy as jnp
from jax import lax
from jax.experimental import pallas as pl
from jax.experimental.pallas import tpu as pltpu
from jax.experimental.pallas import tpu_sc as plsc

NC = 2
NS = 16
NW = NC * NS
LANES = 16


def _compute_y(x, weight, bias, bn):
    n, d_in = x.shape
    t, d_out, _ = weight.shape

    def body(x_ref, w_ref, b_ref, y_ref):
        for ti in range(t):
            y = lax.dot_general(
                x_ref[...], w_ref[ti],
                dimension_numbers=(((1,), (1,)), ((), ())),
                preferred_element_type=jnp.float32,
            )
            y_ref[ti] = y + b_ref[ti]

    return pl.pallas_call(
        body,
        grid=(n // bn,),
        in_specs=[
            pl.BlockSpec((bn, d_in), lambda i: (i, 0)),
            pl.BlockSpec((t, d_out, d_in), lambda i: (0, 0, 0)),
            pl.BlockSpec((t, 1, d_out), lambda i: (0, 0, 0)),
        ],
        out_specs=pl.BlockSpec((t, bn, d_out), lambda i: (0, i, 0)),
        out_shape=jax.ShapeDtypeStruct((t, n, d_out), jnp.float32),
    )(x, weight, bias.reshape(t, 1, d_out))


def _pidx_call(edge_index, edge_type, n_nodes):
    e = edge_type.shape[0]
    rows, cols = e // 1280, 1280
    ei3 = edge_index.reshape(2, rows, cols)
    et2 = edge_type.reshape(rows, cols)

    def body(src_ref, tgt_ref, et_ref, o_ref):
        o_ref[...] = ((et_ref[...] * n_nodes + src_ref[0]) * 16384
                      + tgt_ref[0])

    out = pl.pallas_call(
        body,
        grid=(1,),
        in_specs=[
            pl.BlockSpec((1, rows, cols), lambda i: (0, 0, 0)),
            pl.BlockSpec((1, rows, cols), lambda i: (1, 0, 0)),
            pl.BlockSpec((rows, cols), lambda i: (0, 0)),
        ],
        out_specs=pl.BlockSpec((rows, cols), lambda i: (0, 0)),
        out_shape=jax.ShapeDtypeStruct((rows, cols), jnp.int32),
    )(ei3, ei3, et2)
    return out.reshape(e)


def _add_body(a_ref, b_ref, o_ref):
    o_ref[...] = a_ref[...] + b_ref[...]


def _combine_partials(partial, n, d, bn):
    nb = n // bn
    return pl.pallas_call(
        _add_body,
        grid=(nb,),
        in_specs=[
            pl.BlockSpec((bn, d), lambda i: (i, 0)),
            pl.BlockSpec((bn, d), lambda i: (i + nb, 0)),
        ],
        out_specs=pl.BlockSpec((bn, d), lambda i: (i, 0)),
        out_shape=jax.ShapeDtypeStruct((n, d), jnp.float32),
    )(partial, partial)


def _sc_scatter(y2, pidx, n_nodes, d, ch):
    e = pidx.shape[0]
    epw = e // NW
    nch = epw // ch
    npairs = (nch - 1) // 2
    full = (n_nodes // NS) & ~7
    rem = n_nodes - NS * full
    nz, ztail = full // ch, full % ch
    assert full % 8 == 0 and rem % 8 == 0 and ztail % 8 == 0 and rem <= ch
    assert e % NW == 0 and epw % ch == 0 and ch % LANES == 0 and nch % 2 == 1
    mesh = plsc.VectorSubcoreMesh(
        core_axis_name="c", subcore_axis_name="s", num_cores=NC, num_subcores=NS)

    @functools.partial(
        pl.kernel,
        mesh=mesh,
        out_type=jax.ShapeDtypeStruct((NC * n_nodes, d), jnp.float32),
        scratch_types=[
            pltpu.VMEM((epw,), jnp.int32),
            pltpu.VMEM((ch,), jnp.int32),
            pltpu.VMEM((ch,), jnp.int32),
            pltpu.VMEM((ch,), jnp.int32),
            pltpu.VMEM((ch,), jnp.int32),
            pltpu.VMEM((ch, d), jnp.float32),
            pltpu.VMEM((ch, d), jnp.float32),
            pltpu.VMEM((40, d), jnp.float32),
            pltpu.VMEM_SHARED((n_nodes, d), jnp.float32),
            pltpu.SemaphoreType.DMA,
            pltpu.SemaphoreType.DMA,
            pltpu.SemaphoreType.DMA,
            pltpu.SemaphoreType.DMA,
            pltpu.SemaphoreType.DMA,
            pltpu.SemaphoreType.DMA,
        ],
    )
    def k(y_h, p_h, out_h, pvm, g0, t0, g1, t1, rows0, rows1, zbuf, acc,
          sem0, sem1, ssc0, ssc1, semp, semz):
        c = lax.axis_index("c")
        s = lax.axis_index("s")
        base = (c * NS + s) * epw
        pdma = pltpu.async_copy(p_h.at[pl.ds(base, epw)], pvm, semp)

        zero = jnp.zeros((LANES,), jnp.float32)
        def zrow_body(i, carry):
            for v in range(d // LANES):
                zbuf[i, pl.ds(v * LANES, LANES)] = zero
            return carry
        lax.fori_loop(0, 40, zrow_body, 0)

        def decode(j, gt, tt):
            for v in range(ch // LANES):
                pv = pvm[pl.ds(j * ch + v * LANES, LANES)]
                gt[pl.ds(v * LANES, LANES)] = pv >> 14
                tt[pl.ds(v * LANES, LANES)] = pv & 16383

        a0 = s * full
        nzc, zt = full // 40, full % 40
        for kk in range(nzc):
            pltpu.async_copy(zbuf, acc.at[pl.ds(a0 + kk * 40, 40)], semz)
        if zt:
            pltpu.async_copy(zbuf.at[pl.ds(0, zt)],
                             acc.at[pl.ds(a0 + nzc * 40, zt)], semz)
        @pl.when(s == NS - 1)
        def _zero_tail():
            pltpu.async_copy(zbuf.at[pl.ds(0, rem)],
                             acc.at[pl.ds(NS * full, rem)], semz)

        pdma.wait()
        decode(0, g0, t0)
        pltpu.async_copy(y_h.at[g0], rows0, sem0)
        decode(1, g1, t1)
        pltpu.async_copy(y_h.at[g1], rows1, sem1)

        for kk in range(nzc):
            pltpu.make_async_copy(zbuf, acc.at[pl.ds(a0 + kk * 40, 40)],
                                  semz).wait()
        if zt:
            pltpu.make_async_copy(zbuf.at[pl.ds(0, zt)],
                                  acc.at[pl.ds(a0 + nzc * 40, zt)], semz).wait()
        @pl.when(s == NS - 1)
        def _zero_tail_wait():
            pltpu.make_async_copy(zbuf.at[pl.ds(0, rem)],
                                  acc.at[pl.ds(NS * full, rem)], semz).wait()
        plsc.subcore_barrier()

        pltpu.make_async_copy(y_h.at[g0], rows0, sem0).wait()
        pltpu.async_copy(rows0, acc.at[t0], ssc0, add=True)

        def pair(i, carry):
            j1 = 2 * i + 1
            pltpu.make_async_copy(rows0, acc.at[t0], ssc0).wait()
            decode(j1 + 1, g0, t0)
            pltpu.async_copy(y_h.at[g0], rows0, sem0)
            pltpu.make_async_copy(y_h.at[g1], rows1, sem1).wait()
            pltpu.async_copy(rows1, acc.at[t1], ssc1, add=True)
            pltpu.make_async_copy(rows1, acc.at[t1], ssc1).wait()
            @pl.when(j1 + 2 < nch)
            def _():
                decode(j1 + 2, g1, t1)
                pltpu.async_copy(y_h.at[g1], rows1, sem1)
            pltpu.make_async_copy(y_h.at[g0], rows0, sem0).wait()
            pltpu.async_copy(rows0, acc.at[t0], ssc0, add=True)
            return carry
        lax.fori_loop(0, npairs, pair, 0)
        pltpu.make_async_copy(rows0, acc.at[t0], ssc0).wait()

        plsc.subcore_barrier()
        pltpu.sync_copy(acc.at[pl.ds(a0, full)],
                        out_h.at[pl.ds(c * n_nodes + a0, full)])
        @pl.when(s == NS - 1)
        def _write_tail():
            pltpu.sync_copy(acc.at[pl.ds(NS * full, rem)],
                            out_h.at[pl.ds(c * n_nodes + NS * full, rem)])

    return k(y2, pidx)


def kernel(x, edge_index, edge_type, weight, bias):
    n, d_in = x.shape
    t, d_out, _ = weight.shape
    y = _compute_y(x, weight, bias, bn=1000)
    y2 = y.reshape(t * n, d_out)
    pidx = _pidx_call(edge_index, edge_type, n_nodes=n)
    partial = _sc_scatter(y2, pidx, n_nodes=n, d=d_out, ch=80)
    return _combine_partials(partial, n, d_out, bn=1000)


# speedup vs baseline: 1.6434x; 1.0390x over previous
import functools

import jax
import jax.numpy as jnp
from jax import lax
from jax.experimental import pallas as pl
from jax.experimental.pallas import tpu as pltpu
from jax.experimental.pallas import tpu_sc as plsc

NC = 2
NS = 16
NW = NC * NS
LANES = 16


def _compute_y(x, weight, bias, bn):
    n, d_in = x.shape
    t, d_out, _ = weight.shape

    def body(x_ref, w_ref, b_ref, y_ref):
        for ti in range(t):
            y = lax.dot_general(
                x_ref[...], w_ref[ti],
                dimension_numbers=(((1,), (1,)), ((), ())),
                preferred_element_type=jnp.float32,
            )
            y_ref[ti] = y + b_ref[ti]

    return pl.pallas_call(
        body,
        grid=(n // bn,),
        in_specs=[
            pl.BlockSpec((bn, d_in), lambda i: (i, 0)),
            pl.BlockSpec((t, d_out, d_in), lambda i: (0, 0, 0)),
            pl.BlockSpec((t, 1, d_out), lambda i: (0, 0, 0)),
        ],
        out_specs=pl.BlockSpec((t, bn, d_out), lambda i: (0, i, 0)),
        out_shape=jax.ShapeDtypeStruct((t, n, d_out), jnp.float32),
    )(x, weight, bias.reshape(t, 1, d_out))


def _pidx_call(edge_index, edge_type, n_nodes):
    e = edge_type.shape[0]
    rows, cols = e // 1280, 1280
    ei3 = edge_index.reshape(2, rows, cols)
    et2 = edge_type.reshape(rows, cols)

    def body(src_ref, tgt_ref, et_ref, o_ref):
        o_ref[...] = ((et_ref[...] * n_nodes + src_ref[0]) * 16384
                      + tgt_ref[0])

    out = pl.pallas_call(
        body,
        grid=(1,),
        in_specs=[
            pl.BlockSpec((1, rows, cols), lambda i: (0, 0, 0)),
            pl.BlockSpec((1, rows, cols), lambda i: (1, 0, 0)),
            pl.BlockSpec((rows, cols), lambda i: (0, 0)),
        ],
        out_specs=pl.BlockSpec((rows, cols), lambda i: (0, 0)),
        out_shape=jax.ShapeDtypeStruct((rows, cols), jnp.int32),
    )(ei3, ei3, et2)
    return out.reshape(e)


def _add_body(a_ref, b_ref, o_ref):
    o_ref[...] = a_ref[...] + b_ref[...]


def _combine_partials(partial, n, d, bn):
    nb = n // bn
    return pl.pallas_call(
        _add_body,
        grid=(nb,),
        in_specs=[
            pl.BlockSpec((bn, d), lambda i: (i, 0)),
            pl.BlockSpec((bn, d), lambda i: (i + nb, 0)),
        ],
        out_specs=pl.BlockSpec((bn, d), lambda i: (i, 0)),
        out_shape=jax.ShapeDtypeStruct((n, d), jnp.float32),
    )(partial, partial)


def _sc_scatter(y2, pidx, n_nodes, d, ch):
    e = pidx.shape[0]
    epw = e // NW
    nch = epw // ch
    npairs = (nch - 1) // 2
    full = (n_nodes // NS) & ~7
    rem = n_nodes - NS * full
    nz, ztail = full // ch, full % ch
    assert full % 8 == 0 and rem % 8 == 0 and ztail % 8 == 0 and rem <= ch
    assert e % NW == 0 and epw % ch == 0 and ch % LANES == 0 and nch % 2 == 1
    mesh = plsc.VectorSubcoreMesh(
        core_axis_name="c", subcore_axis_name="s", num_cores=NC, num_subcores=NS)

    @functools.partial(
        pl.kernel,
        mesh=mesh,
        out_type=jax.ShapeDtypeStruct((NC * n_nodes, d), jnp.float32),
        scratch_types=[
            pltpu.VMEM((epw,), jnp.int32),
            pltpu.VMEM((ch,), jnp.int32),
            pltpu.VMEM((ch,), jnp.int32),
            pltpu.VMEM((ch,), jnp.int32),
            pltpu.VMEM((ch,), jnp.int32),
            pltpu.VMEM((ch, d), jnp.float32),
            pltpu.VMEM((ch, d), jnp.float32),
            pltpu.VMEM((40, d), jnp.float32),
            pltpu.VMEM_SHARED((n_nodes, d), jnp.float32),
            pltpu.SemaphoreType.DMA,
            pltpu.SemaphoreType.DMA,
            pltpu.SemaphoreType.DMA,
            pltpu.SemaphoreType.DMA,
            pltpu.SemaphoreType.DMA,
            pltpu.SemaphoreType.DMA,
        ],
    )
    def k(y_h, p_h, out_h, pvm, g0, t0, g1, t1, rows0, rows1, zbuf, acc,
          sem0, sem1, ssc0, ssc1, semp, semz):
        c = lax.axis_index("c")
        s = lax.axis_index("s")
        base = (c * NS + s) * epw
        pdma = pltpu.async_copy(p_h.at[pl.ds(base, epw)], pvm, semp)

        zero = jnp.zeros((LANES,), jnp.float32)
        def zrow_body(i, carry):
            for v in range(d // LANES):
                zbuf[i, pl.ds(v * LANES, LANES)] = zero
            return carry
        lax.fori_loop(0, 40, zrow_body, 0)

        def decode(j, gt, tt):
            for v in range(ch // LANES):
                pv = pvm[pl.ds(j * ch + v * LANES, LANES)]
                gt[pl.ds(v * LANES, LANES)] = pv >> 14
                tt[pl.ds(v * LANES, LANES)] = pv & 16383

        a0 = s * full
        nzc, zt = full // 40, full % 40
        for kk in range(nzc):
            pltpu.async_copy(zbuf, acc.at[pl.ds(a0 + kk * 40, 40)], semz)
        if zt:
            pltpu.async_copy(zbuf.at[pl.ds(0, zt)],
                             acc.at[pl.ds(a0 + nzc * 40, zt)], semz)
        @pl.when(s == NS - 1)
        def _zero_tail():
            pltpu.async_copy(zbuf.at[pl.ds(0, rem)],
                             acc.at[pl.ds(NS * full, rem)], semz)

        pdma.wait()
        decode(0, g0, t0)
        pltpu.async_copy(y_h.at[g0], rows0, sem0)
        decode(1, g1, t1)
        pltpu.async_copy(y_h.at[g1], rows1, sem1)

        for kk in range(nzc):
            pltpu.make_async_copy(zbuf, acc.at[pl.ds(a0 + kk * 40, 40)],
                                  semz).wait()
        if zt:
            pltpu.make_async_copy(zbuf.at[pl.ds(0, zt)],
                                  acc.at[pl.ds(a0 + nzc * 40, zt)], semz).wait()
        @pl.when(s == NS - 1)
        def _zero_tail_wait():
            pltpu.make_async_copy(zbuf.at[pl.ds(0, rem)],
                                  acc.at[pl.ds(NS * full, rem)], semz).wait()
        plsc.subcore_barrier()

        pltpu.make_async_copy(y_h.at[g0], rows0, sem0).wait()
        pltpu.async_copy(rows0, acc.at[t0], ssc0, add=True)

        def pair(i, carry):
            j1 = 2 * i + 1
            pltpu.make_async_copy(rows0, acc.at[t0], ssc0).wait()
            decode(j1 + 1, g0, t0)
            pltpu.async_copy(y_h.at[g0], rows0, sem0)
            pltpu.make_async_copy(y_h.at[g1], rows1, sem1).wait()
            pltpu.async_copy(rows1, acc.at[t1], ssc1, add=True)
            pltpu.make_async_copy(rows1, acc.at[t1], ssc1).wait()
            @pl.when(j1 + 2 < nch)
            def _():
                decode(j1 + 2, g1, t1)
                pltpu.async_copy(y_h.at[g1], rows1, sem1)
            pltpu.make_async_copy(y_h.at[g0], rows0, sem0).wait()
            pltpu.async_copy(rows0, acc.at[t0], ssc0, add=True)
            return carry
        lax.fori_loop(0, npairs, pair, 0)
        pltpu.make_async_copy(rows0, acc.at[t0], ssc0).wait()

        plsc.subcore_barrier()
        pltpu.sync_copy(acc.at[pl.ds(a0, full)],
                        out_h.at[pl.ds(c * n_nodes + a0, full)])
        @pl.when(s == NS - 1)
        def _write_tail():
            pltpu.sync_copy(acc.at[pl.ds(NS * full, rem)],
                            out_h.at[pl.ds(c * n_nodes + NS * full, rem)])

    return k(y2, pidx)


def kernel(x, edge_index, edge_type, weight, bias):
    n, d_in = x.shape
    t, d_out, _ = weight.shape
    y = _compute_y(x, weight, bias, bn=2000)
    y2 = y.reshape(t * n, d_out)
    pidx = _pidx_call(edge_index, edge_type, n_nodes=n)
    partial = _sc_scatter(y2, pidx, n_nodes=n, d=d_out, ch=80)
    return _combine_partials(partial, n, d_out, bn=2000)


# speedup vs baseline: 1.6484x; 1.0030x over previous
import functools

import jax
import jax.numpy as jnp
from jax import lax
from jax.experimental import pallas as pl
from jax.experimental.pallas import tpu as pltpu
from jax.experimental.pallas import tpu_sc as plsc

NC = 2
NS = 16
NW = NC * NS
LANES = 16


def _compute_y(x, weight, bias, bn):
    n, d_in = x.shape
    t, d_out, _ = weight.shape

    def body(x_ref, w_ref, b_ref, y_ref):
        for ti in range(t):
            y = lax.dot_general(
                x_ref[...], w_ref[ti],
                dimension_numbers=(((1,), (1,)), ((), ())),
                preferred_element_type=jnp.float32,
            )
            y_ref[ti] = y + b_ref[ti]

    return pl.pallas_call(
        body,
        grid=(n // bn,),
        in_specs=[
            pl.BlockSpec((bn, d_in), lambda i: (i, 0)),
            pl.BlockSpec((t, d_out, d_in), lambda i: (0, 0, 0)),
            pl.BlockSpec((t, 1, d_out), lambda i: (0, 0, 0)),
        ],
        out_specs=pl.BlockSpec((t, bn, d_out), lambda i: (0, i, 0)),
        out_shape=jax.ShapeDtypeStruct((t, n, d_out), jnp.float32),
    )(x, weight, bias.reshape(t, 1, d_out))


def _pidx_call(edge_index, edge_type, n_nodes):
    e = edge_type.shape[0]
    rows, cols = e // 1280, 1280
    ei3 = edge_index.reshape(2, rows, cols)
    et2 = edge_type.reshape(rows, cols)

    def body(src_ref, tgt_ref, et_ref, o_ref):
        o_ref[...] = ((et_ref[...] * n_nodes + src_ref[0]) * 16384
                      + tgt_ref[0])

    out = pl.pallas_call(
        body,
        grid=(1,),
        in_specs=[
            pl.BlockSpec((1, rows, cols), lambda i: (0, 0, 0)),
            pl.BlockSpec((1, rows, cols), lambda i: (1, 0, 0)),
            pl.BlockSpec((rows, cols), lambda i: (0, 0)),
        ],
        out_specs=pl.BlockSpec((rows, cols), lambda i: (0, 0)),
        out_shape=jax.ShapeDtypeStruct((rows, cols), jnp.int32),
    )(ei3, ei3, et2)
    return out.reshape(e)


def _add_body(a_ref, b_ref, o_ref):
    o_ref[...] = a_ref[...] + b_ref[...]


def _combine_partials(partial, n, d, bn):
    nb = n // bn
    return pl.pallas_call(
        _add_body,
        grid=(nb,),
        in_specs=[
            pl.BlockSpec((bn, d), lambda i: (i, 0)),
            pl.BlockSpec((bn, d), lambda i: (i + nb, 0)),
        ],
        out_specs=pl.BlockSpec((bn, d), lambda i: (i, 0)),
        out_shape=jax.ShapeDtypeStruct((n, d), jnp.float32),
    )(partial, partial)


def _sc_scatter(y2, pidx, n_nodes, d, ch):
    e = pidx.shape[0]
    epw = e // NW
    nch = epw // ch
    npairs = (nch - 1) // 2
    full = (n_nodes // NS) & ~7
    rem = n_nodes - NS * full
    nz, ztail = full // ch, full % ch
    assert full % 8 == 0 and rem % 8 == 0 and ztail % 8 == 0 and rem <= ch
    assert e % NW == 0 and epw % ch == 0 and ch % LANES == 0 and nch % 2 == 1
    mesh = plsc.VectorSubcoreMesh(
        core_axis_name="c", subcore_axis_name="s", num_cores=NC, num_subcores=NS)

    @functools.partial(
        pl.kernel,
        mesh=mesh,
        out_type=jax.ShapeDtypeStruct((NC * n_nodes, d), jnp.float32),
        scratch_types=[
            pltpu.VMEM((epw,), jnp.int32),
            pltpu.VMEM((ch,), jnp.int32),
            pltpu.VMEM((ch,), jnp.int32),
            pltpu.VMEM((ch,), jnp.int32),
            pltpu.VMEM((ch,), jnp.int32),
            pltpu.VMEM((ch, d), jnp.float32),
            pltpu.VMEM((ch, d), jnp.float32),
            pltpu.VMEM((40, d), jnp.float32),
            pltpu.VMEM_SHARED((n_nodes, d), jnp.float32),
            pltpu.SemaphoreType.DMA,
            pltpu.SemaphoreType.DMA,
            pltpu.SemaphoreType.DMA,
            pltpu.SemaphoreType.DMA,
            pltpu.SemaphoreType.DMA,
            pltpu.SemaphoreType.DMA,
        ],
    )
    def k(y_h, p_h, out_h, pvm, g0, t0, g1, t1, rows0, rows1, zbuf, acc,
          sem0, sem1, ssc0, ssc1, semp, semz):
        c = lax.axis_index("c")
        s = lax.axis_index("s")
        base = (c * NS + s) * epw
        pdma = pltpu.async_copy(p_h.at[pl.ds(base, epw)], pvm, semp)

        zero = jnp.zeros((LANES,), jnp.float32)
        def zrow_body(i, carry):
            for v in range(d // LANES):
                zbuf[i, pl.ds(v * LANES, LANES)] = zero
            return carry
        lax.fori_loop(0, 40, zrow_body, 0)

        def decode(j, gt, tt):
            for v in range(ch // LANES):
                pv = pvm[pl.ds(j * ch + v * LANES, LANES)]
                gt[pl.ds(v * LANES, LANES)] = pv >> 14
                tt[pl.ds(v * LANES, LANES)] = pv & 16383

        a0 = s * full
        nzc, zt = full // 40, full % 40
        for kk in range(nzc):
            pltpu.async_copy(zbuf, acc.at[pl.ds(a0 + kk * 40, 40)], semz)
        if zt:
            pltpu.async_copy(zbuf.at[pl.ds(0, zt)],
                             acc.at[pl.ds(a0 + nzc * 40, zt)], semz)
        @pl.when(s == NS - 1)
        def _zero_tail():
            pltpu.async_copy(zbuf.at[pl.ds(0, rem)],
                             acc.at[pl.ds(NS * full, rem)], semz)

        pdma.wait()
        decode(0, g0, t0)
        pltpu.async_copy(y_h.at[g0], rows0, sem0)
        decode(1, g1, t1)
        pltpu.async_copy(y_h.at[g1], rows1, sem1)

        for kk in range(nzc):
            pltpu.make_async_copy(zbuf, acc.at[pl.ds(a0 + kk * 40, 40)],
                                  semz).wait()
        if zt:
            pltpu.make_async_copy(zbuf.at[pl.ds(0, zt)],
                                  acc.at[pl.ds(a0 + nzc * 40, zt)], semz).wait()
        @pl.when(s == NS - 1)
        def _zero_tail_wait():
            pltpu.make_async_copy(zbuf.at[pl.ds(0, rem)],
                                  acc.at[pl.ds(NS * full, rem)], semz).wait()
        plsc.subcore_barrier()

        pltpu.make_async_copy(y_h.at[g0], rows0, sem0).wait()
        pltpu.async_copy(rows0, acc.at[t0], ssc0, add=True)

        def pair(i, carry):
            j1 = 2 * i + 1
            pltpu.make_async_copy(rows0, acc.at[t0], ssc0).wait()
            decode(j1 + 1, g0, t0)
            pltpu.async_copy(y_h.at[g0], rows0, sem0)
            pltpu.make_async_copy(y_h.at[g1], rows1, sem1).wait()
            pltpu.async_copy(rows1, acc.at[t1], ssc1, add=True)
            pltpu.make_async_copy(rows1, acc.at[t1], ssc1).wait()
            @pl.when(j1 + 2 < nch)
            def _():
                decode(j1 + 2, g1, t1)
                pltpu.async_copy(y_h.at[g1], rows1, sem1)
            pltpu.make_async_copy(y_h.at[g0], rows0, sem0).wait()
            pltpu.async_copy(rows0, acc.at[t0], ssc0, add=True)
            return carry
        lax.fori_loop(0, npairs, pair, 0)
        pltpu.make_async_copy(rows0, acc.at[t0], ssc0).wait()

        plsc.subcore_barrier()
        pltpu.sync_copy(acc.at[pl.ds(a0, full)],
                        out_h.at[pl.ds(c * n_nodes + a0, full)])
        @pl.when(s == NS - 1)
        def _write_tail():
            pltpu.sync_copy(acc.at[pl.ds(NS * full, rem)],
                            out_h.at[pl.ds(c * n_nodes + NS * full, rem)])

    return k(y2, pidx)


def kernel(x, edge_index, edge_type, weight, bias):
    n, d_in = x.shape
    t, d_out, _ = weight.shape
    y = _compute_y(x, weight, bias, bn=5000)
    y2 = y.reshape(t * n, d_out)
    pidx = _pidx_call(edge_index, edge_type, n_nodes=n)
    partial = _sc_scatter(y2, pidx, n_nodes=n, d=d_out, ch=80)
    return _combine_partials(partial, n, d_out, bn=5000)


# speedup vs baseline: 1.6509x; 1.0015x over previous
import functools

import jax
import jax.numpy as jnp
from jax import lax
from jax.experimental import pallas as pl
from jax.experimental.pallas import tpu as pltpu
from jax.experimental.pallas import tpu_sc as plsc

NC = 2
NS = 16
NW = NC * NS
LANES = 16


def _compute_y(x, weight, bias, bn):
    n, d_in = x.shape
    t, d_out, _ = weight.shape

    def body(x_ref, w_ref, b_ref, y_ref):
        for ti in range(t):
            y = lax.dot_general(
                x_ref[...], w_ref[ti],
                dimension_numbers=(((1,), (1,)), ((), ())),
                preferred_element_type=jnp.float32,
            )
            y_ref[ti] = y + b_ref[ti]

    return pl.pallas_call(
        body,
        grid=(n // bn,),
        in_specs=[
            pl.BlockSpec((bn, d_in), lambda i: (i, 0)),
            pl.BlockSpec((t, d_out, d_in), lambda i: (0, 0, 0)),
            pl.BlockSpec((t, 1, d_out), lambda i: (0, 0, 0)),
        ],
        out_specs=pl.BlockSpec((t, bn, d_out), lambda i: (0, i, 0)),
        out_shape=jax.ShapeDtypeStruct((t, n, d_out), jnp.float32),
    )(x, weight, bias.reshape(t, 1, d_out))


def _pidx_call(edge_index, edge_type, n_nodes):
    e = edge_type.shape[0]
    rows, cols = e // 1280, 1280
    ei3 = edge_index.reshape(2, rows, cols)
    et2 = edge_type.reshape(rows, cols)

    def body(src_ref, tgt_ref, et_ref, o_ref):
        o_ref[...] = ((et_ref[...] * n_nodes + src_ref[0]) * 16384
                      + tgt_ref[0])

    out = pl.pallas_call(
        body,
        grid=(1,),
        in_specs=[
            pl.BlockSpec((1, rows, cols), lambda i: (0, 0, 0)),
            pl.BlockSpec((1, rows, cols), lambda i: (1, 0, 0)),
            pl.BlockSpec((rows, cols), lambda i: (0, 0)),
        ],
        out_specs=pl.BlockSpec((rows, cols), lambda i: (0, 0)),
        out_shape=jax.ShapeDtypeStruct((rows, cols), jnp.int32),
    )(ei3, ei3, et2)
    return out.reshape(e)


def _add_body(a_ref, b_ref, o_ref):
    o_ref[...] = a_ref[...] + b_ref[...]


def _combine_partials(partial, n, d, bn):
    nb = n // bn
    return pl.pallas_call(
        _add_body,
        grid=(nb,),
        in_specs=[
            pl.BlockSpec((bn, d), lambda i: (i, 0)),
            pl.BlockSpec((bn, d), lambda i: (i + nb, 0)),
        ],
        out_specs=pl.BlockSpec((bn, d), lambda i: (i, 0)),
        out_shape=jax.ShapeDtypeStruct((n, d), jnp.float32),
    )(partial, partial)


def _sc_scatter(y2, pidx, n_nodes, d, ch):
    e = pidx.shape[0]
    epw = e // NW
    nch = epw // ch
    npairs = (nch - 1) // 2
    full = (n_nodes // NS) & ~7
    rem = n_nodes - NS * full
    nz, ztail = full // ch, full % ch
    assert full % 8 == 0 and rem % 8 == 0 and ztail % 8 == 0 and rem <= ch
    assert e % NW == 0 and epw % ch == 0 and ch % LANES == 0 and nch % 2 == 1
    mesh = plsc.VectorSubcoreMesh(
        core_axis_name="c", subcore_axis_name="s", num_cores=NC, num_subcores=NS)

    @functools.partial(
        pl.kernel,
        mesh=mesh,
        out_type=jax.ShapeDtypeStruct((NC * n_nodes, d), jnp.float32),
        compiler_params=pltpu.CompilerParams(needs_layout_passes=False),
        scratch_types=[
            pltpu.VMEM((epw,), jnp.int32),
            pltpu.VMEM((ch,), jnp.int32),
            pltpu.VMEM((ch,), jnp.int32),
            pltpu.VMEM((ch,), jnp.int32),
            pltpu.VMEM((ch,), jnp.int32),
            pltpu.VMEM((ch, d), jnp.float32),
            pltpu.VMEM((ch, d), jnp.float32),
            pltpu.VMEM((40, d), jnp.float32),
            pltpu.VMEM_SHARED((n_nodes, d), jnp.float32),
            pltpu.SemaphoreType.DMA,
            pltpu.SemaphoreType.DMA,
            pltpu.SemaphoreType.DMA,
            pltpu.SemaphoreType.DMA,
            pltpu.SemaphoreType.DMA,
            pltpu.SemaphoreType.DMA,
        ],
    )
    def k(y_h, p_h, out_h, pvm, g0, t0, g1, t1, rows0, rows1, zbuf, acc,
          sem0, sem1, ssc0, ssc1, semp, semz):
        c = lax.axis_index("c")
        s = lax.axis_index("s")
        base = (c * NS + s) * epw
        pdma = pltpu.async_copy(p_h.at[pl.ds(base, epw)], pvm, semp)

        zero = jnp.zeros((LANES,), jnp.float32)
        def zrow_body(i, carry):
            for v in range(d // LANES):
                zbuf[i, pl.ds(v * LANES, LANES)] = zero
            return carry
        lax.fori_loop(0, 40, zrow_body, 0)

        def decode(j, gt, tt):
            for v in range(ch // LANES):
                pv = pvm[pl.ds(j * ch + v * LANES, LANES)]
                gt[pl.ds(v * LANES, LANES)] = pv >> 14
                tt[pl.ds(v * LANES, LANES)] = pv & 16383

        a0 = s * full
        nzc, zt = full // 40, full % 40
        for kk in range(nzc):
            pltpu.async_copy(zbuf, acc.at[pl.ds(a0 + kk * 40, 40)], semz)
        if zt:
            pltpu.async_copy(zbuf.at[pl.ds(0, zt)],
                             acc.at[pl.ds(a0 + nzc * 40, zt)], semz)
        @pl.when(s == NS - 1)
        def _zero_tail():
            pltpu.async_copy(zbuf.at[pl.ds(0, rem)],
                             acc.at[pl.ds(NS * full, rem)], semz)

        pdma.wait()
        decode(0, g0, t0)
        pltpu.async_copy(y_h.at[g0], rows0, sem0)
        decode(1, g1, t1)
        pltpu.async_copy(y_h.at[g1], rows1, sem1)

        for kk in range(nzc):
            pltpu.make_async_copy(zbuf, acc.at[pl.ds(a0 + kk * 40, 40)],
                                  semz).wait()
        if zt:
            pltpu.make_async_copy(zbuf.at[pl.ds(0, zt)],
                                  acc.at[pl.ds(a0 + nzc * 40, zt)], semz).wait()
        @pl.when(s == NS - 1)
        def _zero_tail_wait():
            pltpu.make_async_copy(zbuf.at[pl.ds(0, rem)],
                                  acc.at[pl.ds(NS * full, rem)], semz).wait()
        plsc.subcore_barrier()

        pltpu.make_async_copy(y_h.at[g0], rows0, sem0).wait()
        pltpu.async_copy(rows0, acc.at[t0], ssc0, add=True)

        def pair(i, carry):
            j1 = 2 * i + 1
            pltpu.make_async_copy(rows0, acc.at[t0], ssc0).wait()
            decode(j1 + 1, g0, t0)
            pltpu.async_copy(y_h.at[g0], rows0, sem0)
            pltpu.make_async_copy(y_h.at[g1], rows1, sem1).wait()
            pltpu.async_copy(rows1, acc.at[t1], ssc1, add=True)
            pltpu.make_async_copy(rows1, acc.at[t1], ssc1).wait()
            @pl.when(j1 + 2 < nch)
            def _():
                decode(j1 + 2, g1, t1)
                pltpu.async_copy(y_h.at[g1], rows1, sem1)
            pltpu.make_async_copy(y_h.at[g0], rows0, sem0).wait()
            pltpu.async_copy(rows0, acc.at[t0], ssc0, add=True)
            return carry
        lax.fori_loop(0, npairs, pair, 0)
        pltpu.make_async_copy(rows0, acc.at[t0], ssc0).wait()

        plsc.subcore_barrier()
        pltpu.sync_copy(acc.at[pl.ds(a0, full)],
                        out_h.at[pl.ds(c * n_nodes + a0, full)])
        @pl.when(s == NS - 1)
        def _write_tail():
            pltpu.sync_copy(acc.at[pl.ds(NS * full, rem)],
                            out_h.at[pl.ds(c * n_nodes + NS * full, rem)])

    return k(y2, pidx)


def kernel(x, edge_index, edge_type, weight, bias):
    n, d_in = x.shape
    t, d_out, _ = weight.shape
    y = _compute_y(x, weight, bias, bn=5000)
    y2 = y.reshape(t * n, d_out)
    pidx = _pidx_call(edge_index, edge_type, n_nodes=n)
    partial = _sc_scatter(y2, pidx, n_nodes=n, d=d_out, ch=80)
    return _combine_partials(partial, n, d_out, bn=5000)
